# baseline (device time: 200671 ns/iter reference)
import jax
import jax.numpy as jnp
from jax import lax
from jax.experimental import pallas as pl
from jax.experimental.pallas import tpu as pltpu

T_LOC = 1024
D = 1024
E_LOC = 8
E = 16
F = 4096
F_BLK = 1024
N_FBLK = F // F_BLK
C = 320
TOPK = 2


def _exchange(x, router):

    def body(x_ref, r_ref, xb_ref, xr_ref, gown_ref, gr_ref,
             rr_buf, r16_buf, sems):
        mx = lax.axis_index("x")
        my = lax.axis_index("y")
        mz = lax.axis_index("z")
        partner = (1 - mx, my, mz)

        barrier = pltpu.get_barrier_semaphore()
        pl.semaphore_signal(barrier, inc=1, device_id=partner,
                            device_id_type=pl.DeviceIdType.MESH)
        pl.semaphore_wait(barrier, 1)

        rdma_r = pltpu.make_async_remote_copy(
            src_ref=r_ref, dst_ref=rr_buf, send_sem=sems.at[2],
            recv_sem=sems.at[3],
            device_id=partner, device_id_type=pl.DeviceIdType.MESH)
        rdma_r.start()

        xb_ref[...] = x_ref[...].astype(jnp.bfloat16)
        rdma_x = pltpu.make_async_remote_copy(
            src_ref=xb_ref, dst_ref=xr_ref, send_sem=sems.at[0],
            recv_sem=sems.at[1],
            device_id=partner, device_id_type=pl.DeviceIdType.MESH)
        rdma_x.start()

        rdma_r.wait()

        @pl.when(mx == 0)
        def _():
            r16_buf[:, 0:E_LOC] = r_ref[...]
            r16_buf[:, E_LOC:] = rr_buf[...]

        @pl.when(mx != 0)
        def _():
            r16_buf[:, 0:E_LOC] = rr_buf[...]
            r16_buf[:, E_LOC:] = r_ref[...]

        gown_ref[...] = jnp.dot(x_ref[...], r16_buf[...],
                                precision=lax.Precision.HIGHEST,
                                preferred_element_type=jnp.float32)

        rdma_g = pltpu.make_async_remote_copy(
            src_ref=gown_ref, dst_ref=gr_ref, send_sem=sems.at[4],
            recv_sem=sems.at[5],
            device_id=partner, device_id_type=pl.DeviceIdType.MESH)
        rdma_g.start()
        rdma_g.wait()
        rdma_x.wait()

    return pl.pallas_call(
        body,
        out_shape=(
            jax.ShapeDtypeStruct((T_LOC, D), jnp.bfloat16),
            jax.ShapeDtypeStruct((T_LOC, D), jnp.bfloat16),
            jax.ShapeDtypeStruct((T_LOC, E), jnp.float32),
            jax.ShapeDtypeStruct((T_LOC, E), jnp.float32),
        ),
        in_specs=[pl.BlockSpec(memory_space=pltpu.VMEM)] * 2,
        out_specs=(pl.BlockSpec(memory_space=pltpu.VMEM),) * 4,
        scratch_shapes=[
            pltpu.VMEM((D, E_LOC), jnp.float32),
            pltpu.VMEM((D, E), jnp.float32),
            pltpu.SemaphoreType.DMA((6,)),
        ],
        compiler_params=pltpu.CompilerParams(collective_id=0),
    )(x, router)


def _moe_ffn(x_mine, x_r, idx_c1, idx_1c, wgt_t1, W1, W2):
    T2 = 2 * T_LOC
    bf16 = jnp.bfloat16

    def body(x_ref, xr_ref, ic1_ref, i1c_ref, wgt_ref, w1_ref, w2_ref,
             out_ref, om_ref, ot_ref, xg_ref, yacc_ref, recv_ref, sems):
        e = pl.program_id(0)
        f = pl.program_id(1)
        mx = lax.axis_index("x")
        my = lax.axis_index("y")
        mz = lax.axis_index("z")
        partner = (1 - mx, my, mz)

        @pl.when((e == 0) & (f == 0))
        def _():
            barrier = pltpu.get_barrier_semaphore()
            pl.semaphore_signal(barrier, inc=1, device_id=partner,
                                device_id_type=pl.DeviceIdType.MESH)
            pl.semaphore_wait(barrier, 1)

        @pl.when(f == 0)
        def _():
            iota_t = lax.broadcasted_iota(jnp.int32, (C, T2), 1)
            g = (ic1_ref[0] == iota_t).astype(bf16)
            xg = (jnp.dot(g[:, :T_LOC], x_ref[...],
                          preferred_element_type=jnp.float32)
                  + jnp.dot(g[:, T_LOC:], xr_ref[...],
                            preferred_element_type=jnp.float32))
            xg_ref[...] = xg.astype(bf16)

        h = jnp.maximum(
            jnp.dot(xg_ref[...], w1_ref[0].astype(bf16),
                    preferred_element_type=jnp.float32), 0.0)
        contrib = jnp.dot(h.astype(bf16), w2_ref[0].astype(bf16),
                          preferred_element_type=jnp.float32)

        @pl.when(f == 0)
        def _():
            yacc_ref[...] = contrib

        @pl.when(f != 0)
        def _():
            yacc_ref[...] += contrib

        @pl.when(f == N_FBLK - 1)
        def _():
            yb = yacc_ref[...].astype(bf16)
            iota_lo = lax.broadcasted_iota(jnp.int32, (T_LOC, C), 0)
            gt_lo = ((i1c_ref[0] == iota_lo).astype(bf16)
                     * wgt_ref[0, :T_LOC].astype(bf16))
            mine = jnp.dot(gt_lo, yb, preferred_element_type=jnp.float32)
            gt_hi = ((i1c_ref[0] == iota_lo + T_LOC).astype(bf16)
                     * wgt_ref[0, T_LOC:].astype(bf16))
            theirs = jnp.dot(gt_hi, yb, preferred_element_type=jnp.float32)

            @pl.when(e == 0)
            def _():
                om_ref[...] = mine.astype(bf16)
                ot_ref[...] = theirs.astype(bf16)

            @pl.when(e != 0)
            def _():
                om_ref[...] += mine.astype(bf16)
                ot_ref[...] += theirs.astype(bf16)

        @pl.when((e == E_LOC - 1) & (f == N_FBLK - 1))
        def _():
            rdma = pltpu.make_async_remote_copy(
                src_ref=ot_ref, dst_ref=recv_ref, send_sem=sems.at[0],
                recv_sem=sems.at[1],
                device_id=partner, device_id_type=pl.DeviceIdType.MESH)
            rdma.start()
            rdma.wait()
            out_ref[...] = (om_ref[...].astype(jnp.float32)
                            + recv_ref[...].astype(jnp.float32))

    return pl.pallas_call(
        body,
        grid=(E_LOC, N_FBLK),
        in_specs=[
            pl.BlockSpec((T_LOC, D), lambda e, f: (0, 0)),
            pl.BlockSpec((T_LOC, D), lambda e, f: (0, 0)),
            pl.BlockSpec((1, C, 1), lambda e, f: (e, 0, 0)),
            pl.BlockSpec((1, 1, C), lambda e, f: (e, 0, 0)),
            pl.BlockSpec((1, T2, 1), lambda e, f: (e, 0, 0)),
            pl.BlockSpec((1, D, F_BLK), lambda e, f: (e, 0, f)),
            pl.BlockSpec((1, F_BLK, D), lambda e, f: (e, f, 0)),
        ],
        out_specs=pl.BlockSpec((T_LOC, D), lambda e, f: (0, 0)),
        out_shape=jax.ShapeDtypeStruct((T_LOC, D), jnp.float32),
        scratch_shapes=[
            pltpu.VMEM((T_LOC, D), bf16),
            pltpu.VMEM((T_LOC, D), bf16),
            pltpu.VMEM((C, D), bf16),
            pltpu.VMEM((C, D), jnp.float32),
            pltpu.VMEM((T_LOC, D), bf16),
            pltpu.SemaphoreType.DMA((2,)),
        ],
        compiler_params=pltpu.CompilerParams(
            dimension_semantics=("arbitrary", "arbitrary"),
            collective_id=1,
            vmem_limit_bytes=48 * 1024 * 1024),
    )(x_mine, x_r, idx_c1, idx_1c, wgt_t1, W1, W2)


def kernel(x, router, W1, W2):
    mx = lax.axis_index("x")

    xb, x_r, g_own, g_r = _exchange(x, router)

    gates = jnp.concatenate([g_own, g_r], axis=0)

    top2v, top2i = lax.top_k(gates, TOPK)
    top2v = top2v - top2v.max(axis=1, keepdims=True)
    ew = jnp.exp(top2v)
    w = ew / ew.sum(axis=1, keepdims=True)

    ge = mx * E_LOC + jnp.arange(E_LOC)
    match = top2i[None, :, :] == ge[:, None, None]
    wgt_te = jnp.sum(w[None] * match, axis=-1)
    mask = jnp.any(match, axis=-1)

    idx = jnp.argsort(~mask, axis=1, stable=True)[:, :C]

    return _moe_ffn(
        xb, x_r,
        idx[:, :, None],
        idx[:, None, :],
        wgt_te[:, :, None],
        W1, W2)


# device time: 193455 ns/iter; 1.0373x vs baseline; 1.0373x over previous
import jax
import jax.numpy as jnp
from jax import lax
from jax.experimental import pallas as pl
from jax.experimental.pallas import tpu as pltpu

T_LOC = 1024
D = 1024
E_LOC = 8
E = 16
F = 4096
F_BLK = 1024
N_FBLK = F // F_BLK
C = 320
TOPK = 2


def _exchange(x, router):

    def body(x_ref, r_ref, xb_ref, xr_ref, gown_ref, gr_ref,
             rr_buf, r16_buf, sems):
        mx = lax.axis_index("x")
        my = lax.axis_index("y")
        mz = lax.axis_index("z")
        partner = (1 - mx, my, mz)

        barrier = pltpu.get_barrier_semaphore()
        pl.semaphore_signal(barrier, inc=1, device_id=partner,
                            device_id_type=pl.DeviceIdType.MESH)
        pl.semaphore_wait(barrier, 1)

        rdma_r = pltpu.make_async_remote_copy(
            src_ref=r_ref, dst_ref=rr_buf, send_sem=sems.at[2],
            recv_sem=sems.at[3],
            device_id=partner, device_id_type=pl.DeviceIdType.MESH)
        rdma_r.start()

        xb_ref[...] = x_ref[...].astype(jnp.bfloat16)
        rdma_x = pltpu.make_async_remote_copy(
            src_ref=xb_ref, dst_ref=xr_ref, send_sem=sems.at[0],
            recv_sem=sems.at[1],
            device_id=partner, device_id_type=pl.DeviceIdType.MESH)
        rdma_x.start()

        rdma_r.wait()

        @pl.when(mx == 0)
        def _():
            r16_buf[:, 0:E_LOC] = r_ref[...]
            r16_buf[:, E_LOC:] = rr_buf[...]

        @pl.when(mx != 0)
        def _():
            r16_buf[:, 0:E_LOC] = rr_buf[...]
            r16_buf[:, E_LOC:] = r_ref[...]

        gown_ref[...] = jnp.dot(x_ref[...], r16_buf[...],
                                precision=lax.Precision.HIGHEST,
                                preferred_element_type=jnp.float32)

        rdma_g = pltpu.make_async_remote_copy(
            src_ref=gown_ref, dst_ref=gr_ref, send_sem=sems.at[4],
            recv_sem=sems.at[5],
            device_id=partner, device_id_type=pl.DeviceIdType.MESH)
        rdma_g.start()
        rdma_g.wait()
        rdma_x.wait()

    return pl.pallas_call(
        body,
        out_shape=(
            jax.ShapeDtypeStruct((T_LOC, D), jnp.bfloat16),
            jax.ShapeDtypeStruct((T_LOC, D), jnp.bfloat16),
            jax.ShapeDtypeStruct((T_LOC, E), jnp.float32),
            jax.ShapeDtypeStruct((T_LOC, E), jnp.float32),
        ),
        in_specs=[pl.BlockSpec(memory_space=pltpu.VMEM)] * 2,
        out_specs=(pl.BlockSpec(memory_space=pltpu.VMEM),) * 4,
        scratch_shapes=[
            pltpu.VMEM((D, E_LOC), jnp.float32),
            pltpu.VMEM((D, E), jnp.float32),
            pltpu.SemaphoreType.DMA((6,)),
        ],
        compiler_params=pltpu.CompilerParams(collective_id=0),
    )(x, router)


def _moe_ffn(x_mine, x_r, idx_c1, idx_1c, wgt_t1, W1, W2):
    T2 = 2 * T_LOC
    bf16 = jnp.bfloat16

    def body(x_ref, xr_ref, ic1_ref, i1c_ref, wgt_ref, w1_ref, w2_ref,
             om_ref, ot_ref, xg_ref, yacc_ref):
        e = pl.program_id(0)
        f = pl.program_id(1)

        @pl.when(f == 0)
        def _():
            iota_t = lax.broadcasted_iota(jnp.int32, (C, T2), 1)
            g = (ic1_ref[0] == iota_t).astype(bf16)
            xg = (jnp.dot(g[:, :T_LOC], x_ref[...],
                          preferred_element_type=jnp.float32)
                  + jnp.dot(g[:, T_LOC:], xr_ref[...],
                            preferred_element_type=jnp.float32))
            xg_ref[...] = xg.astype(bf16)

        h = jnp.maximum(
            jnp.dot(xg_ref[...], w1_ref[0].astype(bf16),
                    preferred_element_type=jnp.float32), 0.0)
        contrib = jnp.dot(h.astype(bf16), w2_ref[0].astype(bf16),
                          preferred_element_type=jnp.float32)

        @pl.when(f == 0)
        def _():
            yacc_ref[...] = contrib

        @pl.when(f != 0)
        def _():
            yacc_ref[...] += contrib

        @pl.when(f == N_FBLK - 1)
        def _():
            yb = yacc_ref[...].astype(bf16)
            iota_lo = lax.broadcasted_iota(jnp.int32, (T_LOC, C), 0)
            gt_lo = ((i1c_ref[0] == iota_lo).astype(bf16)
                     * wgt_ref[0, :T_LOC].astype(bf16))
            mine = jnp.dot(gt_lo, yb, preferred_element_type=jnp.float32)
            gt_hi = ((i1c_ref[0] == iota_lo + T_LOC).astype(bf16)
                     * wgt_ref[0, T_LOC:].astype(bf16))
            theirs = jnp.dot(gt_hi, yb, preferred_element_type=jnp.float32)

            @pl.when(e == 0)
            def _():
                om_ref[...] = mine.astype(bf16)
                ot_ref[...] = theirs.astype(bf16)

            @pl.when(e != 0)
            def _():
                om_ref[...] += mine.astype(bf16)
                ot_ref[...] += theirs.astype(bf16)

    return pl.pallas_call(
        body,
        grid=(E_LOC, N_FBLK),
        in_specs=[
            pl.BlockSpec((T_LOC, D), lambda e, f: (0, 0)),
            pl.BlockSpec((T_LOC, D), lambda e, f: (0, 0)),
            pl.BlockSpec((1, C, 1), lambda e, f: (e, 0, 0)),
            pl.BlockSpec((1, 1, C), lambda e, f: (e, 0, 0)),
            pl.BlockSpec((1, T2, 1), lambda e, f: (e, 0, 0)),
            pl.BlockSpec((1, D, F_BLK), lambda e, f: (e, 0, f)),
            pl.BlockSpec((1, F_BLK, D), lambda e, f: (e, f, 0)),
        ],
        out_specs=(
            pl.BlockSpec((T_LOC, D), lambda e, f: (0, 0)),
            pl.BlockSpec((T_LOC, D), lambda e, f: (0, 0)),
        ),
        out_shape=(
            jax.ShapeDtypeStruct((T_LOC, D), bf16),
            jax.ShapeDtypeStruct((T_LOC, D), bf16),
        ),
        scratch_shapes=[
            pltpu.VMEM((C, D), bf16),
            pltpu.VMEM((C, D), jnp.float32),
        ],
        compiler_params=pltpu.CompilerParams(
            dimension_semantics=("arbitrary", "arbitrary")),
    )(x_mine, x_r, idx_c1, idx_1c, wgt_t1, W1, W2)


def _combine(y_mine, y_theirs):

    def body(mine_ref, theirs_ref, out_ref, recv_buf, s_sem, r_sem):
        mx = lax.axis_index("x")
        my = lax.axis_index("y")
        mz = lax.axis_index("z")
        partner = (1 - mx, my, mz)

        barrier = pltpu.get_barrier_semaphore()
        pl.semaphore_signal(barrier, inc=1, device_id=partner,
                            device_id_type=pl.DeviceIdType.MESH)
        pl.semaphore_wait(barrier, 1)

        rdma = pltpu.make_async_remote_copy(
            src_ref=theirs_ref, dst_ref=recv_buf, send_sem=s_sem,
            recv_sem=r_sem,
            device_id=partner, device_id_type=pl.DeviceIdType.MESH)
        rdma.start()
        rdma.wait()
        out_ref[...] = (mine_ref[...].astype(jnp.float32)
                        + recv_buf[...].astype(jnp.float32))

    return pl.pallas_call(
        body,
        out_shape=jax.ShapeDtypeStruct((T_LOC, D), jnp.float32),
        in_specs=[pl.BlockSpec(memory_space=pltpu.VMEM)] * 2,
        out_specs=pl.BlockSpec(memory_space=pltpu.VMEM),
        scratch_shapes=[
            pltpu.VMEM((T_LOC, D), jnp.bfloat16),
            pltpu.SemaphoreType.DMA,
            pltpu.SemaphoreType.DMA,
        ],
        compiler_params=pltpu.CompilerParams(collective_id=1),
    )(y_mine, y_theirs)


def kernel(x, router, W1, W2):
    mx = lax.axis_index("x")

    xb, x_r, g_own, g_r = _exchange(x, router)

    gates = jnp.concatenate([g_own, g_r], axis=0)

    top2v, top2i = lax.top_k(gates, TOPK)
    top2v = top2v - top2v.max(axis=1, keepdims=True)
    ew = jnp.exp(top2v)
    w = ew / ew.sum(axis=1, keepdims=True)

    ge = mx * E_LOC + jnp.arange(E_LOC)
    match = top2i[None, :, :] == ge[:, None, None]
    wgt_te = jnp.sum(w[None] * match, axis=-1)
    mask = jnp.any(match, axis=-1)

    idx = jnp.argsort(~mask, axis=1, stable=True)[:, :C]

    out_mine, out_theirs = _moe_ffn(
        xb, x_r,
        idx[:, :, None],
        idx[:, None, :],
        wgt_te[:, :, None],
        W1, W2)

    return _combine(out_mine, out_theirs)
